# Initial kernel scaffold; baseline (speedup 1.0000x reference)
import jax
import jax.numpy as jnp
from jax.experimental import pallas as pl


def kernel(spatial_feat, structural_feat, neighbor_idx, comb_W, comb_b, comb_gamma, comb_beta, agg_W, agg_b, agg_gamma, agg_beta):
    raise NotImplementedError("write your pallas kernel here")


# speedup vs baseline: 951.9839x; 951.9839x over previous
import functools

import jax
import jax.numpy as jnp
from jax import lax
from jax.experimental import pallas as pl
from jax.experimental.pallas import tpu as pltpu
from jax.experimental.pallas import tpu_sc as plsc

EPS = 1e-5
B = 4
C = 64
N = 50000
BLK = 2048
NB = (N + BLK - 1) // BLK
CHUNK = 2000
NCHUNK = N // CHUNK
T_PER_CHUNK = CHUNK // 16
NC = 2
NS = 16
NW = NC * NS
ROWS_PER_WORKER = (B * C) // NW


def _sc_gather_max_body(st_hbm, idx_hbm, out_hbm, row_a, row_b,
                        idx0, idx1, idx2, out_a, out_b):
    wid = lax.axis_index("s") * NC + lax.axis_index("c")
    b = wid // (NW // B)
    cg = wid % (NW // B)

    def do_pair(c0):
        pltpu.sync_copy(st_hbm.at[pl.ds((b * C + c0) * N, N)], row_a)
        pltpu.sync_copy(st_hbm.at[pl.ds((b * C + c0 + 1) * N, N)], row_b)

        def chunk_body(ci, _):
            base = ci * CHUNK
            for k, ibuf in ((0, idx0), (1, idx1), (2, idx2)):
                pltpu.sync_copy(
                    idx_hbm.at[pl.ds((b * 3 + k) * N + base, CHUNK)], ibuf)

            def t_body(t, _):
                off = t * 16
                i0 = idx0[pl.ds(off, 16)]
                i1 = idx1[pl.ds(off, 16)]
                i2 = idx2[pl.ds(off, 16)]
                for row, obuf in ((row_a, out_a), (row_b, out_b)):
                    g = plsc.load_gather(row, [i0])
                    g = jnp.maximum(g, plsc.load_gather(row, [i1]))
                    g = jnp.maximum(g, plsc.load_gather(row, [i2]))
                    g = jnp.maximum(g, row[pl.ds(base + off, 16)])
                    obuf[pl.ds(off, 16)] = g
                return 0

            lax.fori_loop(0, T_PER_CHUNK, t_body, 0)
            pltpu.sync_copy(out_a,
                            out_hbm.at[pl.ds((b * C + c0) * N + base, CHUNK)])
            pltpu.sync_copy(out_b,
                            out_hbm.at[pl.ds((b * C + c0 + 1) * N + base, CHUNK)])
            return 0

        lax.fori_loop(0, NCHUNK, chunk_body, 0)

    for pair in range(ROWS_PER_WORKER // 2):
        do_pair(cg * ROWS_PER_WORKER + 2 * pair)


def _sc_gather_max(st_flat, idx_flat):
    mesh = plsc.VectorSubcoreMesh(core_axis_name="c", subcore_axis_name="s")
    fn = pl.kernel(
        _sc_gather_max_body,
        out_type=jax.ShapeDtypeStruct((B * C * N,), jnp.float32),
        mesh=mesh,
        compiler_params=pltpu.CompilerParams(needs_layout_passes=False),
        scratch_types=[
            pltpu.VMEM((N,), jnp.float32),
            pltpu.VMEM((N,), jnp.float32),
            pltpu.VMEM((CHUNK,), jnp.int32),
            pltpu.VMEM((CHUNK,), jnp.int32),
            pltpu.VMEM((CHUNK,), jnp.int32),
            pltpu.VMEM((CHUNK,), jnp.float32),
            pltpu.VMEM((CHUNK,), jnp.float32),
        ],
    )
    return fn(st_flat, idx_flat)


def _masked_psums(v, ni):
    lane = lax.broadcasted_iota(jnp.int32, (C, BLK), 1)
    valid = (lane + ni * BLK) < N
    vm = jnp.where(valid, v, 0.0)
    acc1 = jnp.zeros((C, 128), jnp.float32)
    acc2 = jnp.zeros((C, 128), jnp.float32)
    for j in range(BLK // 128):
        chunk = vm[:, j * 128:(j + 1) * 128]
        acc1 = acc1 + chunk
        acc2 = acc2 + chunk * chunk
    return acc1, acc2


def _k1_body(sp_ref, st_ref, w1_ref, w2_ref, b_ref, y_ref, psum_ref):
    bi = pl.program_id(0)
    ni = pl.program_id(1)
    y = jnp.dot(w1_ref[...], sp_ref[0], preferred_element_type=jnp.float32)
    y = y + jnp.dot(w2_ref[...], st_ref[0], preferred_element_type=jnp.float32)
    y = y + b_ref[...]
    y_ref[0] = y

    @pl.when((bi == 0) & (ni == 0))
    def _():
        psum_ref[...] = jnp.zeros_like(psum_ref)

    acc1, acc2 = _masked_psums(y, ni)
    psum_ref[0] += acc1
    psum_ref[1] += acc2


def _k1(spatial, structural, w1, w2, bias):
    return pl.pallas_call(
        _k1_body,
        grid=(B, NB),
        in_specs=[
            pl.BlockSpec((1, C, BLK), lambda b, n: (b, 0, n)),
            pl.BlockSpec((1, C, BLK), lambda b, n: (b, 0, n)),
            pl.BlockSpec((C, C), lambda b, n: (0, 0)),
            pl.BlockSpec((C, C), lambda b, n: (0, 0)),
            pl.BlockSpec((C, 1), lambda b, n: (0, 0)),
        ],
        out_specs=[
            pl.BlockSpec((1, C, BLK), lambda b, n: (b, 0, n)),
            pl.BlockSpec((2, C, 128), lambda b, n: (0, 0, 0)),
        ],
        out_shape=[
            jax.ShapeDtypeStruct((B, C, N), jnp.float32),
            jax.ShapeDtypeStruct((2, C, 128), jnp.float32),
        ],
    )(spatial, structural, w1, w2, bias)


def _k2_body(s_ref, w_ref, b_ref, psum_ref):
    bi = pl.program_id(0)
    ni = pl.program_id(1)
    z = jnp.dot(w_ref[...], s_ref[0], preferred_element_type=jnp.float32)
    z = z + b_ref[...]

    @pl.when((bi == 0) & (ni == 0))
    def _():
        psum_ref[...] = jnp.zeros_like(psum_ref)

    acc1, acc2 = _masked_psums(z, ni)
    psum_ref[0] += acc1
    psum_ref[1] += acc2


def _k2(s, w, bias):
    return pl.pallas_call(
        _k2_body,
        grid=(B, NB),
        in_specs=[
            pl.BlockSpec((1, C, BLK), lambda b, n: (b, 0, n)),
            pl.BlockSpec((C, C), lambda b, n: (0, 0)),
            pl.BlockSpec((C, 1), lambda b, n: (0, 0)),
        ],
        out_specs=pl.BlockSpec((2, C, 128), lambda b, n: (0, 0, 0)),
        out_shape=jax.ShapeDtypeStruct((2, C, 128), jnp.float32),
    )(s, w, bias)


def _k3_body(y_ref, s_ref, sy_ref, oy_ref, wz_ref, bz_ref, out1_ref, out2_ref):
    out1_ref[0] = jnp.maximum(y_ref[0] * sy_ref[...] + oy_ref[...], 0.0)
    z = jnp.dot(wz_ref[...], s_ref[0], preferred_element_type=jnp.float32)
    out2_ref[0] = jnp.maximum(z + bz_ref[...], 0.0)


def _k3(y, s, sy, oy, wz, bz):
    return pl.pallas_call(
        _k3_body,
        grid=(B, NB),
        in_specs=[
            pl.BlockSpec((1, C, BLK), lambda b, n: (b, 0, n)),
            pl.BlockSpec((1, C, BLK), lambda b, n: (b, 0, n)),
            pl.BlockSpec((C, 1), lambda b, n: (0, 0)),
            pl.BlockSpec((C, 1), lambda b, n: (0, 0)),
            pl.BlockSpec((C, C), lambda b, n: (0, 0)),
            pl.BlockSpec((C, 1), lambda b, n: (0, 0)),
        ],
        out_specs=[
            pl.BlockSpec((1, C, BLK), lambda b, n: (b, 0, n)),
            pl.BlockSpec((1, C, BLK), lambda b, n: (b, 0, n)),
        ],
        out_shape=[
            jax.ShapeDtypeStruct((B, C, N), jnp.float32),
            jax.ShapeDtypeStruct((B, C, N), jnp.float32),
        ],
    )(y, s, sy, oy, wz, bz)


def _bn_scale_shift(psum, gamma, beta):
    total = psum.sum(axis=2)
    cnt = float(B * N)
    mean = total[0] / cnt
    var = total[1] / cnt - mean * mean
    scale = gamma * lax.rsqrt(var + EPS)
    shift = beta - mean * scale
    return scale.reshape(C, 1), shift.reshape(C, 1)


def kernel(spatial_feat, structural_feat, neighbor_idx, comb_W, comb_b,
           comb_gamma, comb_beta, agg_W, agg_b, agg_gamma, agg_beta):
    st_flat = structural_feat.reshape(-1)
    idx_flat = jnp.transpose(neighbor_idx, (0, 2, 1)).reshape(-1).astype(jnp.int32)

    s_flat = _sc_gather_max(st_flat, idx_flat)
    s = s_flat.reshape(B, C, N)

    w1 = comb_W[:, :C]
    w2 = comb_W[:, C:]
    y, psum_y = _k1(spatial_feat, structural_feat, w1, w2, comb_b.reshape(C, 1))
    psum_z = _k2(s, agg_W, agg_b.reshape(C, 1))

    sy, oy = _bn_scale_shift(psum_y, comb_gamma, comb_beta)
    sz, oz = _bn_scale_shift(psum_z, agg_gamma, agg_beta)
    wz = agg_W * sz
    bz = agg_b.reshape(C, 1) * sz + oz

    out1, out2 = _k3(y, s, sy, oy, wz, bz)
    return (out1, out2)


# speedup vs baseline: 1516.4116x; 1.5929x over previous
import functools

import jax
import jax.numpy as jnp
from jax import lax
from jax.experimental import pallas as pl
from jax.experimental.pallas import tpu as pltpu
from jax.experimental.pallas import tpu_sc as plsc

EPS = 1e-5
B = 4
C = 64
N = 50000
BLK = 2048
NB = (N + BLK - 1) // BLK
CHUNK = 2000
NCHUNK = N // CHUNK
T_PER_CHUNK = CHUNK // 16
NC = 2
NS = 16
NW = NC * NS
ROWS_PER_WORKER = (B * C) // NW


def _sc_gather_max_body(st_hbm, idx_hbm, out_hbm, row_a, row_b,
                        i0a, i1a, i2a, i0b, i1b, i2b,
                        oa0, oa1, ob0, ob1,
                        sem_row, sem_ia, sem_ib, sem_oa, sem_ob):
    wid = lax.axis_index("s") * NC + lax.axis_index("c")
    b = wid // (NW // B)
    cg = wid % (NW // B)

    idx_sets = ((i0a, i1a, i2a, sem_ia), (i0b, i1b, i2b, sem_ib))
    out_sets = ((oa0, oa1, sem_oa), (ob0, ob1, sem_ob))

    def start_idx(ci, s):
        bufs = idx_sets[s]
        return [
            pltpu.async_copy(
                idx_hbm.at[pl.ds((b * 3 + k) * N + ci * CHUNK, CHUNK)],
                bufs[k], bufs[3])
            for k in range(3)
        ]

    def do_pair(pair, _):
        c0 = cg * ROWS_PER_WORKER + 2 * pair
        rw = [pltpu.async_copy(st_hbm.at[pl.ds((b * C + c0) * N, N)],
                               row_a, sem_row),
              pltpu.async_copy(st_hbm.at[pl.ds((b * C + c0 + 1) * N, N)],
                               row_b, sem_row)]
        pending_idx = start_idx(0, 0)
        for h in rw:
            h.wait()
        pending_out = [None, None]
        for ci in range(NCHUNK):
            s = ci % 2
            cur_idx = pending_idx
            if ci + 1 < NCHUNK:
                pending_idx = start_idx(ci + 1, (ci + 1) % 2)
            for h in cur_idx:
                h.wait()
            if pending_out[s] is not None:
                for h in pending_out[s]:
                    h.wait()
            base = ci * CHUNK
            ib0, ib1, ib2, _ = idx_sets[s]
            obuf_a, obuf_b, sem_o = out_sets[s]

            @plsc.parallel_loop(0, T_PER_CHUNK, unroll=5)
            def _(t, ib0=ib0, ib1=ib1, ib2=ib2,
                  obuf_a=obuf_a, obuf_b=obuf_b, base=base):
                off = t * 16
                i0 = ib0[pl.ds(off, 16)]
                i1 = ib1[pl.ds(off, 16)]
                i2 = ib2[pl.ds(off, 16)]
                for row, obuf in ((row_a, obuf_a), (row_b, obuf_b)):
                    g = plsc.load_gather(row, [i0])
                    g = jnp.maximum(g, plsc.load_gather(row, [i1]))
                    g = jnp.maximum(g, plsc.load_gather(row, [i2]))
                    g = jnp.maximum(g, row[pl.ds(base + off, 16)])
                    obuf[pl.ds(off, 16)] = g

            pending_out[s] = [
                pltpu.async_copy(
                    obuf_a, out_hbm.at[pl.ds((b * C + c0) * N + base, CHUNK)],
                    sem_o),
                pltpu.async_copy(
                    obuf_b,
                    out_hbm.at[pl.ds((b * C + c0 + 1) * N + base, CHUNK)],
                    sem_o)]
        for po in pending_out:
            if po is not None:
                for h in po:
                    h.wait()
        return 0

    lax.fori_loop(0, ROWS_PER_WORKER // 2, do_pair, 0)


def _sc_gather_max(st_flat, idx_flat):
    mesh = plsc.VectorSubcoreMesh(core_axis_name="c", subcore_axis_name="s")
    fn = pl.kernel(
        _sc_gather_max_body,
        out_type=jax.ShapeDtypeStruct((B * C * N,), jnp.float32),
        mesh=mesh,
        compiler_params=pltpu.CompilerParams(needs_layout_passes=False),
        scratch_types=[
            pltpu.VMEM((N,), jnp.float32),
            pltpu.VMEM((N,), jnp.float32),
            pltpu.VMEM((CHUNK,), jnp.int32),
            pltpu.VMEM((CHUNK,), jnp.int32),
            pltpu.VMEM((CHUNK,), jnp.int32),
            pltpu.VMEM((CHUNK,), jnp.int32),
            pltpu.VMEM((CHUNK,), jnp.int32),
            pltpu.VMEM((CHUNK,), jnp.int32),
            pltpu.VMEM((CHUNK,), jnp.float32),
            pltpu.VMEM((CHUNK,), jnp.float32),
            pltpu.VMEM((CHUNK,), jnp.float32),
            pltpu.VMEM((CHUNK,), jnp.float32),
            pltpu.SemaphoreType.DMA,
            pltpu.SemaphoreType.DMA,
            pltpu.SemaphoreType.DMA,
            pltpu.SemaphoreType.DMA,
            pltpu.SemaphoreType.DMA,
        ],
    )
    return fn(st_flat, idx_flat)


def _masked_psums(v, ni):
    lane = lax.broadcasted_iota(jnp.int32, (C, BLK), 1)
    valid = (lane + ni * BLK) < N
    vm = jnp.where(valid, v, 0.0)
    acc1 = jnp.zeros((C, 128), jnp.float32)
    acc2 = jnp.zeros((C, 128), jnp.float32)
    for j in range(BLK // 128):
        chunk = vm[:, j * 128:(j + 1) * 128]
        acc1 = acc1 + chunk
        acc2 = acc2 + chunk * chunk
    return acc1, acc2


def _k1_body(sp_ref, st_ref, w1_ref, w2_ref, b_ref, y_ref, psum_ref):
    bi = pl.program_id(0)
    ni = pl.program_id(1)
    y = jnp.dot(w1_ref[...], sp_ref[0], preferred_element_type=jnp.float32)
    y = y + jnp.dot(w2_ref[...], st_ref[0], preferred_element_type=jnp.float32)
    y = y + b_ref[...]
    y_ref[0] = y

    @pl.when((bi == 0) & (ni == 0))
    def _():
        psum_ref[...] = jnp.zeros_like(psum_ref)

    acc1, acc2 = _masked_psums(y, ni)
    psum_ref[0] += acc1
    psum_ref[1] += acc2


def _k1(spatial, structural, w1, w2, bias):
    return pl.pallas_call(
        _k1_body,
        grid=(B, NB),
        in_specs=[
            pl.BlockSpec((1, C, BLK), lambda b, n: (b, 0, n)),
            pl.BlockSpec((1, C, BLK), lambda b, n: (b, 0, n)),
            pl.BlockSpec((C, C), lambda b, n: (0, 0)),
            pl.BlockSpec((C, C), lambda b, n: (0, 0)),
            pl.BlockSpec((C, 1), lambda b, n: (0, 0)),
        ],
        out_specs=[
            pl.BlockSpec((1, C, BLK), lambda b, n: (b, 0, n)),
            pl.BlockSpec((2, C, 128), lambda b, n: (0, 0, 0)),
        ],
        out_shape=[
            jax.ShapeDtypeStruct((B, C, N), jnp.float32),
            jax.ShapeDtypeStruct((2, C, 128), jnp.float32),
        ],
    )(spatial, structural, w1, w2, bias)


def _k2_body(s_ref, w_ref, b_ref, psum_ref):
    bi = pl.program_id(0)
    ni = pl.program_id(1)
    z = jnp.dot(w_ref[...], s_ref[0], preferred_element_type=jnp.float32)
    z = z + b_ref[...]

    @pl.when((bi == 0) & (ni == 0))
    def _():
        psum_ref[...] = jnp.zeros_like(psum_ref)

    acc1, acc2 = _masked_psums(z, ni)
    psum_ref[0] += acc1
    psum_ref[1] += acc2


def _k2(s, w, bias):
    return pl.pallas_call(
        _k2_body,
        grid=(B, NB),
        in_specs=[
            pl.BlockSpec((1, C, BLK), lambda b, n: (b, 0, n)),
            pl.BlockSpec((C, C), lambda b, n: (0, 0)),
            pl.BlockSpec((C, 1), lambda b, n: (0, 0)),
        ],
        out_specs=pl.BlockSpec((2, C, 128), lambda b, n: (0, 0, 0)),
        out_shape=jax.ShapeDtypeStruct((2, C, 128), jnp.float32),
    )(s, w, bias)


def _k3_body(y_ref, s_ref, sy_ref, oy_ref, wz_ref, bz_ref, out1_ref, out2_ref):
    out1_ref[0] = jnp.maximum(y_ref[0] * sy_ref[...] + oy_ref[...], 0.0)
    z = jnp.dot(wz_ref[...], s_ref[0], preferred_element_type=jnp.float32)
    out2_ref[0] = jnp.maximum(z + bz_ref[...], 0.0)


def _k3(y, s, sy, oy, wz, bz):
    return pl.pallas_call(
        _k3_body,
        grid=(B, NB),
        in_specs=[
            pl.BlockSpec((1, C, BLK), lambda b, n: (b, 0, n)),
            pl.BlockSpec((1, C, BLK), lambda b, n: (b, 0, n)),
            pl.BlockSpec((C, 1), lambda b, n: (0, 0)),
            pl.BlockSpec((C, 1), lambda b, n: (0, 0)),
            pl.BlockSpec((C, C), lambda b, n: (0, 0)),
            pl.BlockSpec((C, 1), lambda b, n: (0, 0)),
        ],
        out_specs=[
            pl.BlockSpec((1, C, BLK), lambda b, n: (b, 0, n)),
            pl.BlockSpec((1, C, BLK), lambda b, n: (b, 0, n)),
        ],
        out_shape=[
            jax.ShapeDtypeStruct((B, C, N), jnp.float32),
            jax.ShapeDtypeStruct((B, C, N), jnp.float32),
        ],
    )(y, s, sy, oy, wz, bz)


def _bn_scale_shift(psum, gamma, beta):
    total = psum.sum(axis=2)
    cnt = float(B * N)
    mean = total[0] / cnt
    var = total[1] / cnt - mean * mean
    scale = gamma * lax.rsqrt(var + EPS)
    shift = beta - mean * scale
    return scale.reshape(C, 1), shift.reshape(C, 1)


def kernel(spatial_feat, structural_feat, neighbor_idx, comb_W, comb_b,
           comb_gamma, comb_beta, agg_W, agg_b, agg_gamma, agg_beta):
    st_flat = structural_feat.reshape(-1)
    idx_flat = jnp.transpose(neighbor_idx, (0, 2, 1)).reshape(-1).astype(jnp.int32)

    s_flat = _sc_gather_max(st_flat, idx_flat)
    s = s_flat.reshape(B, C, N)

    w1 = comb_W[:, :C]
    w2 = comb_W[:, C:]
    y, psum_y = _k1(spatial_feat, structural_feat, w1, w2, comb_b.reshape(C, 1))
    psum_z = _k2(s, agg_W, agg_b.reshape(C, 1))

    sy, oy = _bn_scale_shift(psum_y, comb_gamma, comb_beta)
    sz, oz = _bn_scale_shift(psum_z, agg_gamma, agg_beta)
    wz = agg_W * sz
    bz = agg_b.reshape(C, 1) * sz + oz

    out1, out2 = _k3(y, s, sy, oy, wz, bz)
    return (out1, out2)
